# baseline (device time: 189696 ns/iter reference)
import jax
from jax import lax
from jax.experimental import pallas as pl
from jax.experimental.pallas import tpu as pltpu

NCQ = 8
NH = NCQ // 2


def kernel(x):
    m, n = x.shape
    qrows = m // 4
    rows = qrows // NCQ

    def body(
        x_ref, out_ref, acc, stage,
        ysend, yrecv, fxsend, fxrecv, fzsend, fzrecv,
        dxsend, dxrecv, dzsend, dzrecv, copy_sems, wb_sems,
    ):
        my_x = lax.axis_index("x")
        my_y = lax.axis_index("y")
        my_z = lax.axis_index("z")
        ynbr = (my_x, 1 - my_y, my_z)
        xnbr = (1 - my_x, my_y, my_z)
        znbr = (my_x, my_y, 1 - my_z)

        base_me = (2 * my_x + my_z) * qrows
        base_qx = (2 * (1 - my_x) + my_z) * qrows
        base_qz = (2 * my_x + (1 - my_z)) * qrows
        base_qd = (2 * (1 - my_x) + (1 - my_z)) * qrows

        barrier_sem = pltpu.get_barrier_semaphore()
        for nbr in (ynbr, xnbr, znbr):
            pl.semaphore_signal(
                barrier_sem, inc=1, device_id=nbr,
                device_id_type=pl.DeviceIdType.MESH,
            )
        pl.semaphore_wait(barrier_sem, 3)

        def rdma(ref_base, k, ssem, rsem, dev, src_is_x=False, sem_k=None):
            sl = pl.ds(ref_base + k * rows, rows)
            src = x_ref.at[sl, :] if src_is_x else acc.at[sl, :]
            sk = k if sem_k is None else sem_k
            return pltpu.make_async_remote_copy(
                src_ref=src,
                dst_ref=acc.at[sl, :],
                send_sem=ssem.at[sk],
                recv_sem=rsem.at[sk],
                device_id=dev,
                device_id_type=pl.DeviceIdType.MESH,
            )

        DIAG_Y = [0, 1]
        DIAG_X = [2, 3, 4]
        DIAG_Z = [5, 6, 7]

        ydescs = [
            rdma(base_me, k, ysend, yrecv, ynbr, src_is_x=True)
            for k in range(NCQ)
        ] + [
            rdma(base_qd, k, ysend, yrecv, ynbr, src_is_x=True,
                 sem_k=NCQ + i)
            for i, k in enumerate(DIAG_Y)
        ]
        for d in ydescs:
            d.start()

        fx = [rdma(base_me, k, fxsend, fxrecv, xnbr) for k in range(NCQ)]
        fz = [rdma(base_me, k, fzsend, fzrecv, znbr) for k in range(NCQ)]
        dx = [
            rdma(base_qz, k, dxsend, dxrecv, xnbr, sem_k=i)
            for i, k in enumerate(DIAG_X)
        ]
        dz = [
            rdma(base_qx, k, dzsend, dzrecv, znbr, sem_k=i)
            for i, k in enumerate(DIAG_Z)
        ]

        order = (
            [base_qx, base_qz,
             base_qx + 1 * rows, base_qz + 1 * rows,
             base_qx + 2 * rows, base_qx + 3 * rows, base_qx + 4 * rows,
             base_qz + 5 * rows, base_qz + 6 * rows, base_qz + 7 * rows]
            + [base_me + k * rows for k in range(NCQ)]
            + [base_qd + k * rows for k in DIAG_Y + DIAG_X + DIAG_Z]
            + [base_qz + k * rows for k in DIAG_X]
            + [base_qx + k * rows for k in DIAG_Z]
        )

        def stage_copy(i, slot):
            return pltpu.make_async_copy(
                x_ref.at[pl.ds(order[i], rows), :],
                stage.at[slot],
                copy_sems.at[slot],
            )

        adds_done = [0]

        def do_add():
            i = adds_done[0]
            adds_done[0] = i + 1
            if i + 1 < len(order):
                stage_copy(i + 1, (i + 1) % 2).start()
            stage_copy(i, i % 2).wait()
            sl = pl.ds(order[i], rows)
            acc[sl, :] = acc[sl, :] + stage[i % 2]
            pltpu.make_async_copy(
                acc.at[sl, :], out_ref.at[sl, :], wb_sems.at[i]
            ).start()

        stage_copy(0, 0).start()

        LAG = 2
        for k in range(NCQ + LAG):
            if k < NCQ:
                ydescs[k].wait_recv()
                fx[k].start()
                fz[k].start()
            j = k - LAG
            if 0 <= j:
                fx[j].wait_recv()
                if j in DIAG_Z:
                    dz[DIAG_Z.index(j)].start()
                else:
                    do_add()
                fz[j].wait_recv()
                if j in DIAG_X:
                    dx[DIAG_X.index(j)].start()
                else:
                    do_add()

        for k in range(NCQ):
            fx[k].wait_send()
            fz[k].wait_send()
            do_add()

        for i in range(len(DIAG_Y)):
            ydescs[NCQ + i].wait_recv()
            do_add()
        for i in range(len(DIAG_X)):
            dx[i].wait_recv()
            do_add()
        for i in range(len(DIAG_Z)):
            dz[i].wait_recv()
            do_add()

        for i in range(len(DIAG_X)):
            dx[i].wait_send()
            do_add()
        for i in range(len(DIAG_Z)):
            dz[i].wait_send()
            do_add()
        for d in ydescs:
            d.wait_send()
        for i in range(len(order)):
            sl = pl.ds(order[i], rows)
            pltpu.make_async_copy(
                acc.at[sl, :], out_ref.at[sl, :], wb_sems.at[i]
            ).wait()

    return pl.pallas_call(
        body,
        out_shape=jax.ShapeDtypeStruct((m, n), x.dtype),
        in_specs=[pl.BlockSpec(memory_space=pl.ANY)],
        out_specs=pl.BlockSpec(memory_space=pl.ANY),
        scratch_shapes=[
            pltpu.VMEM((m, n), x.dtype),
            pltpu.VMEM((2, rows, n), x.dtype),
            pltpu.SemaphoreType.DMA((NCQ + 2,)),
            pltpu.SemaphoreType.DMA((NCQ + 2,)),
            pltpu.SemaphoreType.DMA((NCQ,)),
            pltpu.SemaphoreType.DMA((NCQ,)),
            pltpu.SemaphoreType.DMA((NCQ,)),
            pltpu.SemaphoreType.DMA((NCQ,)),
            pltpu.SemaphoreType.DMA((3,)),
            pltpu.SemaphoreType.DMA((3,)),
            pltpu.SemaphoreType.DMA((3,)),
            pltpu.SemaphoreType.DMA((3,)),
            pltpu.SemaphoreType.DMA((2,)),
            pltpu.SemaphoreType.DMA((4 * NCQ,)),
        ],
        compiler_params=pltpu.CompilerParams(
            collective_id=0,
            vmem_limit_bytes=60 * 1024 * 1024,
        ),
    )(x)


# device time: 177061 ns/iter; 1.0714x vs baseline; 1.0714x over previous
import jax
from jax import lax
from jax.experimental import pallas as pl
from jax.experimental.pallas import tpu as pltpu

NCQ = 8
NH = NCQ // 2


def kernel(x):
    m, n = x.shape
    qrows = m // 4
    rows = qrows // NCQ

    def body(
        x_ref, out_ref, acc, stage,
        ysend, yrecv, fxsend, fxrecv, fzsend, fzrecv,
        dxsend, dxrecv, dzsend, dzrecv, copy_sems, wb_sems,
    ):
        my_x = lax.axis_index("x")
        my_y = lax.axis_index("y")
        my_z = lax.axis_index("z")
        ynbr = (my_x, 1 - my_y, my_z)
        xnbr = (1 - my_x, my_y, my_z)
        znbr = (my_x, my_y, 1 - my_z)

        base_me = (2 * my_x + my_z) * qrows
        base_qx = (2 * (1 - my_x) + my_z) * qrows
        base_qz = (2 * my_x + (1 - my_z)) * qrows
        base_qd = (2 * (1 - my_x) + (1 - my_z)) * qrows

        barrier_sem = pltpu.get_barrier_semaphore()
        for nbr in (ynbr, xnbr, znbr):
            pl.semaphore_signal(
                barrier_sem, inc=1, device_id=nbr,
                device_id_type=pl.DeviceIdType.MESH,
            )
        pl.semaphore_wait(barrier_sem, 3)

        def rdma(ref_base, k, ssem, rsem, dev, src_is_x=False, sem_k=None):
            sl = pl.ds(ref_base + k * rows, rows)
            src = x_ref.at[sl, :] if src_is_x else acc.at[sl, :]
            sk = k if sem_k is None else sem_k
            return pltpu.make_async_remote_copy(
                src_ref=src,
                dst_ref=acc.at[sl, :],
                send_sem=ssem.at[sk],
                recv_sem=rsem.at[sk],
                device_id=dev,
                device_id_type=pl.DeviceIdType.MESH,
            )

        DIAG_Y = [0, 1]
        DIAG_X = [2, 3, 4]
        DIAG_Z = [5, 6, 7]

        ydescs = [
            rdma(base_me, k, ysend, yrecv, ynbr, src_is_x=True)
            for k in range(NCQ)
        ] + [
            rdma(base_qd, k, ysend, yrecv, ynbr, src_is_x=True,
                 sem_k=NCQ + i)
            for i, k in enumerate(DIAG_Y)
        ]
        for d in ydescs:
            d.start()

        fx = [rdma(base_me, k, fxsend, fxrecv, xnbr) for k in range(NCQ)]
        fz = [rdma(base_me, k, fzsend, fzrecv, znbr) for k in range(NCQ)]
        dx = [
            rdma(base_qz, k, dxsend, dxrecv, xnbr, sem_k=i)
            for i, k in enumerate(DIAG_X)
        ]
        dz = [
            rdma(base_qx, k, dzsend, dzrecv, znbr, sem_k=i)
            for i, k in enumerate(DIAG_Z)
        ]

        order = (
            [base_qx, base_qz,
             base_qx + 1 * rows, base_qz + 1 * rows,
             base_qx + 2 * rows, base_qx + 3 * rows, base_qx + 4 * rows,
             base_qz + 5 * rows, base_qz + 6 * rows, base_qz + 7 * rows]
            + [base_me + k * rows for k in range(NCQ)]
            + [base_qd + k * rows for k in DIAG_Y + DIAG_X + DIAG_Z]
            + [base_qz + k * rows for k in DIAG_X]
            + [base_qx + k * rows for k in DIAG_Z]
        )

        def stage_copy(i, slot):
            return pltpu.make_async_copy(
                x_ref.at[pl.ds(order[i], rows), :],
                stage.at[slot],
                copy_sems.at[slot],
            )

        adds_done = [0]

        def do_add():
            i = adds_done[0]
            adds_done[0] = i + 1
            if i + 1 < len(order):
                stage_copy(i + 1, (i + 1) % 2).start()
            stage_copy(i, i % 2).wait()
            sl = pl.ds(order[i], rows)
            acc[sl, :] = acc[sl, :] + stage[i % 2]
            pltpu.make_async_copy(
                acc.at[sl, :], out_ref.at[sl, :], wb_sems.at[i]
            ).start()

        stage_copy(0, 0).start()

        for k in range(NCQ):
            ydescs[k].wait_recv()
            fx[k].start()
            fz[k].start()

        for k in range(NCQ):
            fx[k].wait_recv()
            if k in DIAG_Z:
                dz[DIAG_Z.index(k)].start()
            else:
                do_add()
            fz[k].wait_recv()
            if k in DIAG_X:
                dx[DIAG_X.index(k)].start()
            else:
                do_add()

        for k in range(NCQ):
            fx[k].wait_send()
            fz[k].wait_send()
            do_add()

        for i in range(len(DIAG_Y)):
            ydescs[NCQ + i].wait_recv()
            do_add()
        for i in range(len(DIAG_X)):
            dx[i].wait_recv()
            do_add()
        for i in range(len(DIAG_Z)):
            dz[i].wait_recv()
            do_add()

        for i in range(len(DIAG_X)):
            dx[i].wait_send()
            do_add()
        for i in range(len(DIAG_Z)):
            dz[i].wait_send()
            do_add()
        for d in ydescs:
            d.wait_send()
        for i in range(len(order)):
            sl = pl.ds(order[i], rows)
            pltpu.make_async_copy(
                acc.at[sl, :], out_ref.at[sl, :], wb_sems.at[i]
            ).wait()

    return pl.pallas_call(
        body,
        out_shape=jax.ShapeDtypeStruct((m, n), x.dtype),
        in_specs=[pl.BlockSpec(memory_space=pl.ANY)],
        out_specs=pl.BlockSpec(memory_space=pl.ANY),
        scratch_shapes=[
            pltpu.VMEM((m, n), x.dtype),
            pltpu.VMEM((2, rows, n), x.dtype),
            pltpu.SemaphoreType.DMA((NCQ + 2,)),
            pltpu.SemaphoreType.DMA((NCQ + 2,)),
            pltpu.SemaphoreType.DMA((NCQ,)),
            pltpu.SemaphoreType.DMA((NCQ,)),
            pltpu.SemaphoreType.DMA((NCQ,)),
            pltpu.SemaphoreType.DMA((NCQ,)),
            pltpu.SemaphoreType.DMA((3,)),
            pltpu.SemaphoreType.DMA((3,)),
            pltpu.SemaphoreType.DMA((3,)),
            pltpu.SemaphoreType.DMA((3,)),
            pltpu.SemaphoreType.DMA((2,)),
            pltpu.SemaphoreType.DMA((4 * NCQ,)),
        ],
        compiler_params=pltpu.CompilerParams(
            collective_id=0,
            vmem_limit_bytes=60 * 1024 * 1024,
        ),
    )(x)
